# baseline (speedup 1.0000x reference)
import functools

import jax
import jax.numpy as jnp
from jax import lax
from jax.experimental import pallas as pl
from jax.experimental.pallas import tpu as pltpu
from jax.experimental.pallas import tpu_sc as plsc

_NC = 2
_NS = 16
_NW = _NC * _NS
_CHUNK = 50
_G = 40
_NBUF = 4
_RPT = 632
_NPAD = _NS * _RPT
_DEGPT = 640
_DEGPAD = _NS * _DEGPT


def _mesh():
    return plsc.VectorSubcoreMesh(core_axis_name="c", subcore_axis_name="s")


@functools.lru_cache(maxsize=None)
def _make_spmm(n, e, f, with_deg):
    ept = e // _NW
    nch = ept // _CHUNK
    ngrp = nch // _G

    out_type = jax.ShapeDtypeStruct((_NC, _NPAD, f), jnp.float32)
    scratch = [
        pltpu.VMEM((_G, _CHUNK), jnp.int32),
        pltpu.VMEM((_G, _CHUNK), jnp.int32),
        pltpu.VMEM((_G, _CHUNK), jnp.int32),
        pltpu.VMEM((_G, _CHUNK), jnp.int32),
    ] + [pltpu.VMEM((_NBUF * _CHUNK, f), jnp.float32)
    ] + [pltpu.SemaphoreType.DMA
         for _ in range(2 * _NBUF + 2)
    ] + [pltpu.VMEM_SHARED((_NPAD, f), jnp.float32)]
    if with_deg:
        out_type = (out_type,
                    jax.ShapeDtypeStruct((_NC * _DEGPAD,), jnp.float32))
        scratch = scratch + [
            pltpu.VMEM((_DEGPT + 128,), jnp.float32),
            pltpu.VMEM_SHARED((_DEGPAD,), jnp.float32),
        ]

    @functools.partial(
        pl.kernel, out_type=out_type, mesh=_mesh(), scratch_types=scratch)
    def spmm(src3_hbm, dst3_hbm, z_hbm, *refs):
        if with_deg:
            out_hbm, deg_hbm = refs[0], refs[1]
            refs = refs[2:]
        else:
            out_hbm = refs[0]
            refs = refs[1:]
        sb0, sb1, db0, db1 = refs[0:4]
        rowsbuf = refs[4]
        rows = tuple(rowsbuf.at[pl.ds(b * _CHUNK, _CHUNK)]
                     for b in range(_NBUF))
        gsem = refs[5:5 + _NBUF]
        ssem = refs[5 + _NBUF:5 + 2 * _NBUF]
        is0, is1 = refs[5 + 2 * _NBUF:7 + 2 * _NBUF]
        acc_sh = refs[7 + 2 * _NBUF]
        if with_deg:
            dzo_v, dacc_sh = refs[8 + 2 * _NBUF:]
        c = lax.axis_index("c")
        s = lax.axis_index("s")
        wid = s * _NC + c
        sbufs = (sb0, sb1)
        dbufs = (db0, db1)
        isems = (is0, is1)

        def idx_start(g):
            pltpu.async_copy(src3_hbm.at[wid, pl.ds(g * _G, _G)],
                             sbufs[g % 2], isems[g % 2])
            pltpu.async_copy(dst3_hbm.at[wid, pl.ds(g * _G, _G)],
                             dbufs[g % 2], isems[g % 2])

        def idx_wait(g):
            pltpu.make_async_copy(src3_hbm.at[wid, pl.ds(g * _G, _G)],
                                  sbufs[g % 2], isems[g % 2]).wait()
            pltpu.make_async_copy(dst3_hbm.at[wid, pl.ds(g * _G, _G)],
                                  dbufs[g % 2], isems[g % 2]).wait()

        idx_start(0)

        def zstore(i, _):
            rowsbuf[i // (f // 16), pl.ds((i % (f // 16)) * 16, 16)] = (
                jnp.zeros((16,), jnp.float32))
            return 0
        lax.fori_loop(0, _CHUNK * (f // 16), zstore, 0)
        base = s * _RPT
        zds = []
        for r in range(_RPT // _CHUNK):
            zds.append(pltpu.async_copy(
                rowsbuf.at[pl.ds(0, _CHUNK)],
                acc_sh.at[pl.ds(base + r * _CHUNK, _CHUNK)], ssem[0]))
        rem = _RPT - (_RPT // _CHUNK) * _CHUNK
        if rem:
            zds.append(pltpu.async_copy(
                rowsbuf.at[pl.ds(0, rem)],
                acc_sh.at[pl.ds(base + (_RPT // _CHUNK) * _CHUNK, rem)],
                ssem[0]))
        if with_deg:
            for i in range(128 // 16):
                dzo_v[pl.ds(_DEGPT + i * 16, 16)] = jnp.ones((16,),
                                                             jnp.float32)

            def dzstore(i, _):
                dzo_v[pl.ds(i * 16, 16)] = jnp.zeros((16,), jnp.float32)
                return 0
            lax.fori_loop(0, _DEGPT // 16, dzstore, 0)
            zds.append(pltpu.async_copy(
                dzo_v.at[pl.ds(0, _DEGPT)],
                dacc_sh.at[pl.ds(s * _DEGPT, _DEGPT)], ssem[1]))
        for d in zds:
            d.wait()
        plsc.subcore_barrier()

        def wait_g(sbuf, k, buf, sem):
            pltpu.make_async_copy(z_hbm.at[sbuf.at[k]], buf, sem).wait()

        def scat(buf, dbuf, k, sem):
            d = pltpu.async_copy(buf, acc_sh.at[dbuf.at[k]], sem, add=True)
            if with_deg:
                dd = pltpu.async_copy(dzo_v.at[pl.ds(_DEGPT, _CHUNK)],
                                      dacc_sh.at[dbuf.at[k]], sem, add=True)
                return (d, dd)
            return (d,)

        idx_wait(0)
        for b in range(_NBUF):
            pltpu.async_copy(z_hbm.at[sb0.at[b]], rows[b], gsem[b])

        for g in range(ngrp):
            sbuf, dbuf = sbufs[g % 2], dbufs[g % 2]
            if g + 1 < ngrp:
                idx_start(g + 1)

            def wave(w, _, sbuf=sbuf, dbuf=dbuf):
                k = w * _NBUF
                ds = []
                for b in range(_NBUF):
                    wait_g(sbuf, k + b, rows[b], gsem[b])
                    ds.append(scat(rows[b], dbuf, k + b, ssem[b]))
                for b in range(_NBUF):
                    for d in ds[b]:
                        d.wait()
                    pltpu.async_copy(z_hbm.at[sbuf.at[k + _NBUF + b]],
                                     rows[b], gsem[b])
                return 0
            lax.fori_loop(0, _G // _NBUF - 1, wave, 0)

            k = _G - _NBUF
            ds = []
            for b in range(_NBUF):
                wait_g(sbuf, k + b, rows[b], gsem[b])
                ds.append(scat(rows[b], dbuf, k + b, ssem[b]))
            if g + 1 < ngrp:
                idx_wait(g + 1)
                for b in range(_NBUF):
                    for d in ds[b]:
                        d.wait()
                    pltpu.async_copy(z_hbm.at[sbufs[(g + 1) % 2].at[b]],
                                     rows[b], gsem[b])
            else:
                for dd in ds:
                    for d in dd:
                        d.wait()
        plsc.subcore_barrier()

        if with_deg:
            dd = pltpu.async_copy(
                dacc_sh.at[pl.ds(s * _DEGPT, _DEGPT)],
                deg_hbm.at[pl.ds(c * _DEGPAD + s * _DEGPT, _DEGPT)], ssem[1])
            pltpu.sync_copy(acc_sh.at[pl.ds(base, _RPT)],
                            out_hbm.at[c, pl.ds(base, _RPT)])
            dd.wait()
        else:
            pltpu.sync_copy(acc_sh.at[pl.ds(base, _RPT)],
                            out_hbm.at[c, pl.ds(base, _RPT)])

    return spmm


_BM = 5000


@functools.lru_cache(maxsize=None)
def _make_layer(n, k, f, act):
    kp = max(k, 128)
    def body(h_ref, wt_ref, wb_ref, b_ref, s_ref, d_ref, o_ref):
        dsum = d_ref[0] + d_ref[1]
        inv = 1.0 / jnp.maximum(dsum, 1.0)
        mean = (s_ref[0, :, :k] + s_ref[1, :, :k]) * inv
        r = (jnp.dot(h_ref[...], wt_ref[...],
                     preferred_element_type=jnp.float32)
             + jnp.dot(mean, wb_ref[...],
                       preferred_element_type=jnp.float32)
             + b_ref[...])
        o_ref[...] = jnp.maximum(r, 0.0) if act else r
    return pl.pallas_call(
        body,
        grid=(n // _BM,),
        in_specs=[
            pl.BlockSpec((_BM, k), lambda i: (i, 0)),
            pl.BlockSpec((k, f), lambda i: (0, 0)),
            pl.BlockSpec((k, f), lambda i: (0, 0)),
            pl.BlockSpec((1, f), lambda i: (0, 0)),
            pl.BlockSpec((_NC, _BM, kp), lambda i: (0, i, 0)),
            pl.BlockSpec((_NC, _BM, 1), lambda i: (0, i, 0)),
        ],
        out_specs=pl.BlockSpec((_BM, f), lambda i: (i, 0)),
        out_shape=jax.ShapeDtypeStruct((n, f), jnp.float32),
    )


def kernel(x, adj, W1, b1, W2, b2, W3, b3, W4, b4):
    n = x.shape[0]
    e = adj.shape[1]
    nch = e // (_NW * _CHUNK)
    src3 = adj[0].reshape(_NW, nch, _CHUNK)
    dst3 = adj[1].reshape(_NW, nch, _CHUNK)

    h = x
    deg3 = None
    for i, (W, b, act) in enumerate(((W1, b1, True), (W2, b2, True),
                                     (W3, b3, True), (W4, b4, False))):
        k = W.shape[0] // 2
        f = W.shape[1]
        if i == 0:
            s2, deg_flat = _make_spmm(n, e, k, True)(src3, dst3, h)
            deg3 = deg_flat.reshape(_NC, _DEGPAD, 1)
        else:
            s2 = _make_spmm(n, e, k, False)(src3, dst3, h)
        h = _make_layer(n, k, f, act)(
            h, W[:k], W[k:], b.reshape(1, f), s2, deg3)
    return h


# speedup vs baseline: 1.0032x; 1.0032x over previous
import functools

import jax
import jax.numpy as jnp
from jax import lax
from jax.experimental import pallas as pl
from jax.experimental.pallas import tpu as pltpu
from jax.experimental.pallas import tpu_sc as plsc

_NC = 2
_NS = 16
_NW = _NC * _NS
_CHUNK = 50
_G = 40
_NBUF = 4
_RPT = 632
_NPAD = _NS * _RPT
_DEGPT = 640
_DEGPAD = _NS * _DEGPT


def _mesh():
    return plsc.VectorSubcoreMesh(core_axis_name="c", subcore_axis_name="s")


@functools.lru_cache(maxsize=None)
def _make_spmm(n, e, f, with_deg):
    ept = e // _NW
    nch = ept // _CHUNK
    ngrp = nch // _G

    out_type = jax.ShapeDtypeStruct((_NC, _NPAD, f), jnp.float32)
    scratch = [
        pltpu.VMEM((_G, _CHUNK), jnp.int32),
        pltpu.VMEM((_G, _CHUNK), jnp.int32),
        pltpu.VMEM((_G, _CHUNK), jnp.int32),
        pltpu.VMEM((_G, _CHUNK), jnp.int32),
    ] + [pltpu.VMEM((_NBUF * _CHUNK, f), jnp.float32)
    ] + [pltpu.SemaphoreType.DMA
         for _ in range(2 * _NBUF + 2)
    ] + [pltpu.VMEM_SHARED((_NPAD, f), jnp.float32)]
    if with_deg:
        out_type = (out_type,
                    jax.ShapeDtypeStruct((_NC * _DEGPAD,), jnp.float32))
        scratch = scratch + [
            pltpu.VMEM((_DEGPT + 128,), jnp.float32),
            pltpu.VMEM_SHARED((_DEGPAD,), jnp.float32),
        ]

    @functools.partial(
        pl.kernel, out_type=out_type, mesh=_mesh(), scratch_types=scratch)
    def spmm(src3_hbm, dst3_hbm, z_hbm, *refs):
        if with_deg:
            out_hbm, deg_hbm = refs[0], refs[1]
            refs = refs[2:]
        else:
            out_hbm = refs[0]
            refs = refs[1:]
        sb0, sb1, db0, db1 = refs[0:4]
        rowsbuf = refs[4]
        rows = tuple(rowsbuf.at[pl.ds(b * _CHUNK, _CHUNK)]
                     for b in range(_NBUF))
        gsem = refs[5:5 + _NBUF]
        ssem = refs[5 + _NBUF:5 + 2 * _NBUF]
        is0, is1 = refs[5 + 2 * _NBUF:7 + 2 * _NBUF]
        acc_sh = refs[7 + 2 * _NBUF]
        if with_deg:
            dzo_v, dacc_sh = refs[8 + 2 * _NBUF:]
        c = lax.axis_index("c")
        s = lax.axis_index("s")
        wid = s * _NC + c
        sbufs = (sb0, sb1)
        dbufs = (db0, db1)
        isems = (is0, is1)

        def idx_start(g):
            pltpu.async_copy(src3_hbm.at[wid, pl.ds(g * _G, _G)],
                             sbufs[g % 2], isems[g % 2])
            pltpu.async_copy(dst3_hbm.at[wid, pl.ds(g * _G, _G)],
                             dbufs[g % 2], isems[g % 2])

        def idx_wait(g):
            pltpu.make_async_copy(src3_hbm.at[wid, pl.ds(g * _G, _G)],
                                  sbufs[g % 2], isems[g % 2]).wait()
            pltpu.make_async_copy(dst3_hbm.at[wid, pl.ds(g * _G, _G)],
                                  dbufs[g % 2], isems[g % 2]).wait()

        idx_start(0)

        def zstore(i, _):
            rowsbuf[i // (f // 16), pl.ds((i % (f // 16)) * 16, 16)] = (
                jnp.zeros((16,), jnp.float32))
            return 0
        lax.fori_loop(0, _CHUNK * (f // 16), zstore, 0)
        base = s * _RPT
        zds = []
        for r in range(_RPT // _CHUNK):
            zds.append(pltpu.async_copy(
                rowsbuf.at[pl.ds(0, _CHUNK)],
                acc_sh.at[pl.ds(base + r * _CHUNK, _CHUNK)], ssem[0]))
        rem = _RPT - (_RPT // _CHUNK) * _CHUNK
        if rem:
            zds.append(pltpu.async_copy(
                rowsbuf.at[pl.ds(0, rem)],
                acc_sh.at[pl.ds(base + (_RPT // _CHUNK) * _CHUNK, rem)],
                ssem[0]))
        if with_deg:
            for i in range(128 // 16):
                dzo_v[pl.ds(_DEGPT + i * 16, 16)] = jnp.ones((16,),
                                                             jnp.float32)

            def dzstore(i, _):
                dzo_v[pl.ds(i * 16, 16)] = jnp.zeros((16,), jnp.float32)
                return 0
            lax.fori_loop(0, _DEGPT // 16, dzstore, 0)
            zds.append(pltpu.async_copy(
                dzo_v.at[pl.ds(0, _DEGPT)],
                dacc_sh.at[pl.ds(s * _DEGPT, _DEGPT)], ssem[1]))
        for d in zds:
            d.wait()
        plsc.subcore_barrier()

        def wait_g(sbuf, k, buf, sem):
            pltpu.make_async_copy(z_hbm.at[sbuf.at[k]], buf, sem).wait()

        def scat(buf, dbuf, k, sem):
            d = pltpu.async_copy(buf, acc_sh.at[dbuf.at[k]], sem, add=True)
            if with_deg:
                dd = pltpu.async_copy(dzo_v.at[pl.ds(_DEGPT, _CHUNK)],
                                      dacc_sh.at[dbuf.at[k]], sem, add=True)
                return (d, dd)
            return (d,)

        idx_wait(0)
        for b in range(_NBUF):
            pltpu.async_copy(z_hbm.at[sb0.at[b]], rows[b], gsem[b])

        for g in range(ngrp):
            sbuf, dbuf = sbufs[g % 2], dbufs[g % 2]
            if g + 1 < ngrp:
                idx_start(g + 1)

            def wave(w, _, sbuf=sbuf, dbuf=dbuf):
                k = w * _NBUF
                ds = []
                for b in range(_NBUF):
                    wait_g(sbuf, k + b, rows[b], gsem[b])
                    ds.append(scat(rows[b], dbuf, k + b, ssem[b]))
                for b in range(_NBUF):
                    for d in ds[b]:
                        d.wait()
                    pltpu.async_copy(z_hbm.at[sbuf.at[k + _NBUF + b]],
                                     rows[b], gsem[b])
                return 0
            lax.fori_loop(0, _G // _NBUF - 1, wave, 0)

            k = _G - _NBUF
            ds = []
            for b in range(_NBUF):
                wait_g(sbuf, k + b, rows[b], gsem[b])
                ds.append(scat(rows[b], dbuf, k + b, ssem[b]))
            if g + 1 < ngrp:
                idx_wait(g + 1)
                for b in range(_NBUF):
                    for d in ds[b]:
                        d.wait()
                    pltpu.async_copy(z_hbm.at[sbufs[(g + 1) % 2].at[b]],
                                     rows[b], gsem[b])
            else:
                for dd in ds:
                    for d in dd:
                        d.wait()
        plsc.subcore_barrier()

        if with_deg:
            dd = pltpu.async_copy(
                dacc_sh.at[pl.ds(s * _DEGPT, _DEGPT)],
                deg_hbm.at[pl.ds(c * _DEGPAD + s * _DEGPT, _DEGPT)], ssem[1])
            pltpu.sync_copy(acc_sh.at[pl.ds(base, _RPT)],
                            out_hbm.at[c, pl.ds(base, _RPT)])
            dd.wait()
        else:
            pltpu.sync_copy(acc_sh.at[pl.ds(base, _RPT)],
                            out_hbm.at[c, pl.ds(base, _RPT)])

    return spmm


_BM = 2000


@functools.lru_cache(maxsize=None)
def _make_layer(n, k, f, act):
    kp = max(k, 128)
    def body(h_ref, wt_ref, wb_ref, b_ref, s_ref, d_ref, o_ref):
        dsum = d_ref[0] + d_ref[1]
        inv = 1.0 / jnp.maximum(dsum, 1.0)
        mean = (s_ref[0, :, :k] + s_ref[1, :, :k]) * inv
        r = (jnp.dot(h_ref[...], wt_ref[...],
                     preferred_element_type=jnp.float32)
             + jnp.dot(mean, wb_ref[...],
                       preferred_element_type=jnp.float32)
             + b_ref[...])
        o_ref[...] = jnp.maximum(r, 0.0) if act else r
    return pl.pallas_call(
        body,
        grid=(n // _BM,),
        in_specs=[
            pl.BlockSpec((_BM, k), lambda i: (i, 0)),
            pl.BlockSpec((k, f), lambda i: (0, 0)),
            pl.BlockSpec((k, f), lambda i: (0, 0)),
            pl.BlockSpec((1, f), lambda i: (0, 0)),
            pl.BlockSpec((_NC, _BM, kp), lambda i: (0, i, 0)),
            pl.BlockSpec((_NC, _BM, 1), lambda i: (0, i, 0)),
        ],
        out_specs=pl.BlockSpec((_BM, f), lambda i: (i, 0)),
        out_shape=jax.ShapeDtypeStruct((n, f), jnp.float32),
    )


def kernel(x, adj, W1, b1, W2, b2, W3, b3, W4, b4):
    n = x.shape[0]
    e = adj.shape[1]
    nch = e // (_NW * _CHUNK)
    src3 = adj[0].reshape(_NW, nch, _CHUNK)
    dst3 = adj[1].reshape(_NW, nch, _CHUNK)

    h = x
    deg3 = None
    for i, (W, b, act) in enumerate(((W1, b1, True), (W2, b2, True),
                                     (W3, b3, True), (W4, b4, False))):
        k = W.shape[0] // 2
        f = W.shape[1]
        if i == 0:
            s2, deg_flat = _make_spmm(n, e, k, True)(src3, dst3, h)
            deg3 = deg_flat.reshape(_NC, _DEGPAD, 1)
        else:
            s2 = _make_spmm(n, e, k, False)(src3, dst3, h)
        h = _make_layer(n, k, f, act)(
            h, W[:k], W[k:], b.reshape(1, f), s2, deg3)
    return h
